# baseline (speedup 1.0000x reference)
import functools

import jax
import jax.numpy as jnp
from jax import lax
from jax.experimental import pallas as pl
from jax.experimental.pallas import tpu as pltpu
from jax.experimental.pallas import tpu_sc as plsc

_TOPK = 16
_SEG = 128
_NW = 32
_IDX_CHUNK = 128


def _dist_segmin_body(qsq_ref, ksq_ref, q_ref, k_ref,
                      dist_out, gid_out, seg_out, m_scratch,
                      *, block_q, block_k, n_seg):
    j = pl.program_id(0)
    i = pl.program_id(1)
    nk = pl.num_programs(0)
    seg_in_blk = block_k // _SEG

    qk = lax.dot_general(q_ref[...], k_ref[...], (((1,), (1,)), ((), ())),
                         preferred_element_type=jnp.float32)
    d2 = (qsq_ref[...] + ksq_ref[...]) - 2.0 * qk
    dist = jnp.sqrt(jnp.clip(d2, 1e-12, None))
    dist3 = jnp.reshape(dist, (block_q, seg_in_blk, _SEG))
    dist_out[...] = dist3

    m_scratch[j, i] = jnp.min(dist3, axis=-1)

    @pl.when(j == nk - 1)
    def _select_segments():
        m3 = m_scratch[:, i]
        sid3 = (lax.broadcasted_iota(jnp.int32, m3.shape, 0) * seg_in_blk
                + lax.broadcasted_iota(jnp.int32, m3.shape, 2))
        row_ids = (i * block_q
                   + lax.broadcasted_iota(jnp.int32, (block_q, 1), 0))
        for t in range(_TOPK):
            mv = jnp.min(jnp.min(m3, axis=0), axis=-1, keepdims=True)
            sel = jnp.min(jnp.min(jnp.where(m3 == mv[None], sid3, n_seg),
                                  axis=0),
                          axis=-1, keepdims=True)
            gid_out[:, t:t + 1] = row_ids * n_seg + sel
            seg_out[:, t:t + 1] = sel
            if t + 1 < _TOPK:
                m3 = jnp.where(sid3 == sel[None], jnp.inf, m3)


def _sc_gather(table_hbm, gid_hbm, out_hbm, idx_v, rows_v, sem):
    wid = lax.axis_index("s") * 2 + lax.axis_index("c")
    chunks = gid_hbm.shape[1]
    base = wid * chunks * _IDX_CHUNK
    pltpu.sync_copy(gid_hbm.at[wid], idx_v)
    for b in range(chunks):
        pltpu.async_copy(table_hbm.at[idx_v.at[b]], rows_v, sem).wait()
        pltpu.sync_copy(rows_v,
                        out_hbm.at[pl.ds(base + b * _IDX_CHUNK, _IDX_CHUNK)])


def _final_body(seg_ref, cand_ref, dist_out, idx_out, *, block_q):
    v3 = jnp.reshape(cand_ref[...], (block_q, _TOPK, _SEG))
    lane3 = lax.broadcasted_iota(jnp.int32, (block_q, _TOPK, _SEG), 2)
    c3 = seg_ref[...][:, :, None] * _SEG + lane3
    big = jnp.int32(1 << 30)
    for t in range(_TOPK):
        mv = jnp.min(jnp.min(v3, axis=2), axis=1, keepdims=True)
        eq = v3 == mv[:, :, None]
        sel = jnp.min(jnp.min(jnp.where(eq, c3, big), axis=2),
                      axis=1, keepdims=True)
        dist_out[:, t:t + 1] = mv
        idx_out[:, t:t + 1] = sel
        if t + 1 < _TOPK:
            v3 = jnp.where(c3 == sel[:, :, None], jnp.inf, v3)


def kernel(queries, keys, k):
    q_n, d = queries.shape
    k_n, _ = keys.shape
    bq = 512 if q_n % 512 == 0 else q_n
    bk = 2048 if k_n % 2048 == 0 else k_n
    n_seg = k_n // _SEG

    q_sq = jnp.sum(queries * queries, axis=1, keepdims=True)
    k_sq = jnp.sum(keys * keys, axis=1)[None, :]

    dist3, gid, seg = pl.pallas_call(
        functools.partial(_dist_segmin_body, block_q=bq, block_k=bk,
                          n_seg=n_seg),
        grid=(k_n // bk, q_n // bq),
        in_specs=[
            pl.BlockSpec((bq, 1), lambda j, i: (i, 0)),
            pl.BlockSpec((1, bk), lambda j, i: (0, j)),
            pl.BlockSpec((bq, d), lambda j, i: (i, 0)),
            pl.BlockSpec((bk, d), lambda j, i: (j, 0)),
        ],
        out_specs=[
            pl.BlockSpec((bq, bk // _SEG, _SEG), lambda j, i: (i, j, 0)),
            pl.BlockSpec((bq, _TOPK), lambda j, i: (i, 0)),
            pl.BlockSpec((bq, _TOPK), lambda j, i: (i, 0)),
        ],
        out_shape=[
            jax.ShapeDtypeStruct((q_n, n_seg, _SEG), jnp.float32),
            jax.ShapeDtypeStruct((q_n, _TOPK), jnp.int32),
            jax.ShapeDtypeStruct((q_n, _TOPK), jnp.int32),
        ],
        scratch_shapes=[pltpu.VMEM((k_n // bk, q_n // bq, bq, bk // _SEG),
                                   jnp.float32)],
        compiler_params=pltpu.CompilerParams(
            dimension_semantics=("arbitrary", "arbitrary")),
    )(q_sq, k_sq, queries, keys)

    n_gather = q_n * _TOPK
    chunks = n_gather // (_NW * _IDX_CHUNK)
    table = dist3.reshape(q_n * n_seg, _SEG)
    gid_tiled = gid.reshape(_NW, chunks, _IDX_CHUNK)

    sc_gather = pl.kernel(
        _sc_gather,
        mesh=plsc.VectorSubcoreMesh(core_axis_name="c", subcore_axis_name="s"),
        out_type=jax.ShapeDtypeStruct((n_gather, _SEG), jnp.float32),
        scratch_types=[
            pltpu.VMEM((chunks, _IDX_CHUNK), jnp.int32),
            pltpu.VMEM((_IDX_CHUNK, _SEG), jnp.float32),
            pltpu.SemaphoreType.DMA,
        ],
    )
    cand = sc_gather(table, gid_tiled)

    bq3 = 256 if q_n % 256 == 0 else q_n
    nn_dist, nn_idx = pl.pallas_call(
        functools.partial(_final_body, block_q=bq3),
        grid=(q_n // bq3,),
        in_specs=[
            pl.BlockSpec((bq3, _TOPK), lambda i: (i, 0)),
            pl.BlockSpec((bq3 * _TOPK, _SEG), lambda i: (i, 0)),
        ],
        out_specs=[
            pl.BlockSpec((bq3, _TOPK), lambda i: (i, 0)),
            pl.BlockSpec((bq3, _TOPK), lambda i: (i, 0)),
        ],
        out_shape=[
            jax.ShapeDtypeStruct((q_n, _TOPK), jnp.float32),
            jax.ShapeDtypeStruct((q_n, _TOPK), jnp.int32),
        ],
    )(seg, cand)
    return nn_dist, nn_idx + (k - _TOPK)


# speedup vs baseline: 1.0120x; 1.0120x over previous
import functools

import jax
import jax.numpy as jnp
from jax import lax
from jax.experimental import pallas as pl
from jax.experimental.pallas import tpu as pltpu
from jax.experimental.pallas import tpu_sc as plsc

_TOPK = 16
_SEG = 128
_NW = 32
_IDX_CHUNK = 128


def _dist_segmin_body(qsq_ref, ksq_ref, q_ref, k_ref,
                      dist_out, gid_out, seg_out, m_scratch,
                      *, block_q, block_k, n_seg):
    j = pl.program_id(0)
    i = pl.program_id(1)
    nk = pl.num_programs(0)
    seg_in_blk = block_k // _SEG

    qk = lax.dot_general(q_ref[...], k_ref[...], (((1,), (1,)), ((), ())),
                         preferred_element_type=jnp.float32)
    d2 = (qsq_ref[...] + ksq_ref[...]) - 2.0 * qk
    dist = jnp.sqrt(jnp.clip(d2, 1e-12, None))
    dist3 = jnp.reshape(dist, (block_q, seg_in_blk, _SEG))
    dist_out[...] = dist3

    m_scratch[j, i] = jnp.min(dist3, axis=-1)

    @pl.when(j == nk - 1)
    def _select_segments():
        m3 = m_scratch[:, i]
        sid3 = (lax.broadcasted_iota(jnp.int32, m3.shape, 0) * seg_in_blk
                + lax.broadcasted_iota(jnp.int32, m3.shape, 2))
        row_ids = (i * block_q
                   + lax.broadcasted_iota(jnp.int32, (block_q, 1), 0))
        for t in range(_TOPK):
            mv = jnp.min(jnp.min(m3, axis=0), axis=-1, keepdims=True)
            sel = jnp.min(jnp.min(jnp.where(m3 == mv[None], sid3, n_seg),
                                  axis=0),
                          axis=-1, keepdims=True)
            gid_out[:, t:t + 1] = row_ids * n_seg + sel
            seg_out[:, t:t + 1] = sel
            if t + 1 < _TOPK:
                m3 = jnp.where(sid3 == sel[None], jnp.inf, m3)


def _sc_gather(table_hbm, gid_hbm, out_hbm, idx_v, rows_v, sem):
    wid = lax.axis_index("s") * 2 + lax.axis_index("c")
    chunks = gid_hbm.shape[1]
    base = wid * chunks * _IDX_CHUNK
    pltpu.sync_copy(gid_hbm.at[wid], idx_v)
    for b in range(chunks):
        pltpu.async_copy(table_hbm.at[idx_v.at[b]], rows_v, sem).wait()
        pltpu.sync_copy(rows_v,
                        out_hbm.at[pl.ds(base + b * _IDX_CHUNK, _IDX_CHUNK)])


def _final_body(seg_ref, cand_ref, dist_out, idx_out, *, block_q):
    v3 = jnp.reshape(cand_ref[...], (block_q, _TOPK, _SEG))
    lane3 = lax.broadcasted_iota(jnp.int32, (block_q, _TOPK, _SEG), 2)
    c3 = seg_ref[...][:, :, None] * _SEG + lane3
    big = jnp.int32(1 << 30)
    for t in range(_TOPK):
        mv = jnp.min(jnp.min(v3, axis=2), axis=1, keepdims=True)
        eq = v3 == mv[:, :, None]
        sel = jnp.min(jnp.min(jnp.where(eq, c3, big), axis=2),
                      axis=1, keepdims=True)
        dist_out[:, t:t + 1] = mv
        idx_out[:, t:t + 1] = sel
        if t + 1 < _TOPK:
            v3 = jnp.where(c3 == sel[:, :, None], jnp.inf, v3)


def kernel(queries, keys, k):
    q_n, d = queries.shape
    k_n, _ = keys.shape
    bq = 512 if q_n % 512 == 0 else q_n
    bk = 4096 if k_n % 4096 == 0 else k_n
    n_seg = k_n // _SEG

    q_sq = jnp.sum(queries * queries, axis=1, keepdims=True)
    k_sq = jnp.sum(keys * keys, axis=1)[None, :]

    dist3, gid, seg = pl.pallas_call(
        functools.partial(_dist_segmin_body, block_q=bq, block_k=bk,
                          n_seg=n_seg),
        grid=(k_n // bk, q_n // bq),
        in_specs=[
            pl.BlockSpec((bq, 1), lambda j, i: (i, 0)),
            pl.BlockSpec((1, bk), lambda j, i: (0, j)),
            pl.BlockSpec((bq, d), lambda j, i: (i, 0)),
            pl.BlockSpec((bk, d), lambda j, i: (j, 0)),
        ],
        out_specs=[
            pl.BlockSpec((bq, bk // _SEG, _SEG), lambda j, i: (i, j, 0)),
            pl.BlockSpec((bq, _TOPK), lambda j, i: (i, 0)),
            pl.BlockSpec((bq, _TOPK), lambda j, i: (i, 0)),
        ],
        out_shape=[
            jax.ShapeDtypeStruct((q_n, n_seg, _SEG), jnp.float32),
            jax.ShapeDtypeStruct((q_n, _TOPK), jnp.int32),
            jax.ShapeDtypeStruct((q_n, _TOPK), jnp.int32),
        ],
        scratch_shapes=[pltpu.VMEM((k_n // bk, q_n // bq, bq, bk // _SEG),
                                   jnp.float32)],
        compiler_params=pltpu.CompilerParams(
            dimension_semantics=("arbitrary", "arbitrary")),
    )(q_sq, k_sq, queries, keys)

    n_gather = q_n * _TOPK
    chunks = n_gather // (_NW * _IDX_CHUNK)
    table = dist3.reshape(q_n * n_seg, _SEG)
    gid_tiled = gid.reshape(_NW, chunks, _IDX_CHUNK)

    sc_gather = pl.kernel(
        _sc_gather,
        mesh=plsc.VectorSubcoreMesh(core_axis_name="c", subcore_axis_name="s"),
        out_type=jax.ShapeDtypeStruct((n_gather, _SEG), jnp.float32),
        scratch_types=[
            pltpu.VMEM((chunks, _IDX_CHUNK), jnp.int32),
            pltpu.VMEM((_IDX_CHUNK, _SEG), jnp.float32),
            pltpu.SemaphoreType.DMA,
        ],
    )
    cand = sc_gather(table, gid_tiled)

    bq3 = 512 if q_n % 512 == 0 else q_n
    nn_dist, nn_idx = pl.pallas_call(
        functools.partial(_final_body, block_q=bq3),
        grid=(q_n // bq3,),
        in_specs=[
            pl.BlockSpec((bq3, _TOPK), lambda i: (i, 0)),
            pl.BlockSpec((bq3 * _TOPK, _SEG), lambda i: (i, 0)),
        ],
        out_specs=[
            pl.BlockSpec((bq3, _TOPK), lambda i: (i, 0)),
            pl.BlockSpec((bq3, _TOPK), lambda i: (i, 0)),
        ],
        out_shape=[
            jax.ShapeDtypeStruct((q_n, _TOPK), jnp.float32),
            jax.ShapeDtypeStruct((q_n, _TOPK), jnp.int32),
        ],
    )(seg, cand)
    return nn_dist, nn_idx + (k - _TOPK)
